# baseline (speedup 1.0000x reference)
import jax
import jax.numpy as jnp
from jax.experimental import pallas as pl
from jax.experimental.pallas import tpu as pltpu

_BM = 512


def _outer(x_hbm, w_ref, o_hbm):
    m, k = x_hbm.shape
    e = w_ref.shape[0]

    def _inner(x_blk, o_blk):
        o_blk[...] = jax.lax.dot_general(
            x_blk[...], w_ref[...],
            dimension_numbers=(((1,), (1,)), ((), ())),
            preferred_element_type=jnp.float32,
        )

    pltpu.emit_pipeline(
        _inner,
        grid=(m // _BM,),
        in_specs=[
            pl.BlockSpec((_BM, k), lambda i: (i, 0),
                         pipeline_mode=pl.Buffered(buffer_count=4)),
        ],
        out_specs=[
            pl.BlockSpec((_BM, e), lambda i: (i, 0)),
        ],
    )(x_hbm, o_hbm)


def kernel(hidden_states, weight):
    m, k = hidden_states.shape
    e = weight.shape[0]
    return pl.pallas_call(
        _outer,
        in_specs=[
            pl.BlockSpec(memory_space=pltpu.HBM),
            pl.BlockSpec(memory_space=pltpu.VMEM),
        ],
        out_specs=pl.BlockSpec(memory_space=pltpu.HBM),
        out_shape=jax.ShapeDtypeStruct((m, e), jnp.float32),
    )(hidden_states, weight)
